# baseline (speedup 1.0000x reference)
import functools

import jax
import jax.numpy as jnp
from jax.experimental import pallas as pl
from jax.experimental.pallas import tpu as pltpu


def _lstm_body(xa_ref, xb_ref, wx_ref, wh_ref, b_ref, out_ref, cf_ref, hf_ref,
               xpa_ref, xpb_ref, hs_ref, c_ref, h_ref, *, T, mem, batch, U):
    k = pl.program_id(0)
    G = pl.num_programs(0)

    @pl.when(k == 0)
    def _():
        c_ref[:] = jnp.zeros_like(c_ref)
        h_ref[:] = jnp.zeros_like(h_ref)
        xt = xa_ref[:].transpose(1, 0, 2).reshape(T * batch, -1)
        xpa_ref[:] = jnp.dot(xt, wx_ref[:],
                             preferred_element_type=jnp.float32) + b_ref[:]

    def one_step(src_ref, r, c, h):
        z = src_ref[pl.ds(r, batch), :] + jnp.dot(
            h, wh_ref[:], preferred_element_type=jnp.float32)
        ti = jnp.tanh(z[:, :mem])
        to = jnp.tanh(z[:, mem:2 * mem])
        tu = jnp.tanh(z[:, 2 * mem:3 * mem])
        tf = jnp.tanh(z[:, 3 * mem:])
        c = 0.25 * (ti * tu + ti + tu + 1.0) + 0.5 * (tf * c + c)
        tc = jnp.tanh(c)
        h = 0.5 * (to * tc + tc)
        hs_ref[pl.ds(r, batch), :] = h
        return c, h

    def run_loop(src_ref, dst_ref):
        def iter_body(i, carry):
            c, h = carry
            base = (T - 1 - i * U) * batch
            for u in range(U):
                c, h = one_step(src_ref, base - u * batch, c, h)
            xn = xb_ref[:, pl.ds(i * U, U), :].transpose(1, 0, 2).reshape(
                U * batch, -1)
            dst_ref[pl.ds(i * U * batch, U * batch), :] = jnp.dot(
                xn, wx_ref[:], preferred_element_type=jnp.float32) + b_ref[:]
            return (c, h)

        c, h = jax.lax.fori_loop(0, T // U, iter_body, (c_ref[:], h_ref[:]))
        c_ref[:] = c
        h_ref[:] = h

    @pl.when(jax.lax.rem(k, 2) == 0)
    def _():
        run_loop(xpa_ref, xpb_ref)

    @pl.when(jax.lax.rem(k, 2) == 1)
    def _():
        run_loop(xpb_ref, xpa_ref)

    out_ref[:] = hs_ref[:].reshape(T, batch, mem).transpose(1, 0, 2)

    @pl.when(k == G - 1)
    def _():
        cf_ref[:] = c_ref[:]
        hf_ref[:] = h_ref[:]


def kernel(trees, inputs, ioux_w, ioux_b, iouh_w, iouh_b, fx_w, fx_b, fh_w, fh_b):
    del trees
    b, n, in_dim = inputs.shape
    mem = fx_b.shape[0]
    wx = 0.5 * jnp.concatenate([ioux_w, fx_w], axis=0).T
    wh = 0.5 * jnp.concatenate([iouh_w, fh_w], axis=0).T
    bias = 0.5 * jnp.concatenate([ioux_b + iouh_b, fx_b + fh_b])[None, :]

    T = 128
    U = 32
    G = n // T
    body = functools.partial(_lstm_body, T=T, mem=mem, batch=b, U=U)
    o_states, c_fin, h_fin = pl.pallas_call(
        body,
        grid=(G,),
        in_specs=[
            pl.BlockSpec((b, T, in_dim),
                         lambda k: (0, pl.num_programs(0) - 1 - k, 0)),
            pl.BlockSpec((b, T, in_dim),
                         lambda k: (0, jnp.maximum(pl.num_programs(0) - 2 - k, 0), 0)),
            pl.BlockSpec((in_dim, 4 * mem), lambda k: (0, 0)),
            pl.BlockSpec((mem, 4 * mem), lambda k: (0, 0)),
            pl.BlockSpec((1, 4 * mem), lambda k: (0, 0)),
        ],
        out_specs=[
            pl.BlockSpec((b, T, mem),
                         lambda k: (0, pl.num_programs(0) - 1 - k, 0)),
            pl.BlockSpec((b, mem), lambda k: (0, 0)),
            pl.BlockSpec((b, mem), lambda k: (0, 0)),
        ],
        out_shape=[
            jax.ShapeDtypeStruct((b, n, mem), jnp.float32),
            jax.ShapeDtypeStruct((b, mem), jnp.float32),
            jax.ShapeDtypeStruct((b, mem), jnp.float32),
        ],
        scratch_shapes=[
            pltpu.VMEM((T * b, 4 * mem), jnp.float32),
            pltpu.VMEM((T * b, 4 * mem), jnp.float32),
            pltpu.VMEM((T * b, mem), jnp.float32),
            pltpu.VMEM((b, mem), jnp.float32),
            pltpu.VMEM((b, mem), jnp.float32),
        ],
    )(inputs, inputs, wx, wh, bias)
    return (o_states, c_fin, h_fin)


# speedup vs baseline: 1.0193x; 1.0193x over previous
import functools

import jax
import jax.numpy as jnp
from jax.experimental import pallas as pl
from jax.experimental.pallas import tpu as pltpu


def _lstm_body(x_ref, wx_ref, wh_ref, b_ref, out_ref, cf_ref, hf_ref,
               xp_ref, hs_ref, c_ref, h_ref, *, T, mem, batch):
    k = pl.program_id(0)
    G = pl.num_programs(0)

    @pl.when(k == 0)
    def _():
        c_ref[:] = jnp.zeros_like(c_ref)
        h_ref[:] = jnp.zeros_like(h_ref)

    xt = x_ref[:].transpose(1, 0, 2).reshape(T * batch, -1)
    xp_ref[:] = jnp.dot(xt, wx_ref[:], preferred_element_type=jnp.float32) + b_ref[:]

    def one_step(r, c, h):
        z = xp_ref[pl.ds(r, batch), :] + jnp.dot(
            h, wh_ref[:], preferred_element_type=jnp.float32)
        ti = jnp.tanh(z[:, :mem])
        to = jnp.tanh(z[:, mem:2 * mem])
        tu = jnp.tanh(z[:, 2 * mem:3 * mem])
        tf = jnp.tanh(z[:, 3 * mem:])
        c = 0.25 * (ti * tu + ti + tu + 1.0) + 0.5 * (tf * c + c)
        tc = jnp.tanh(c)
        h = 0.5 * (to * tc + tc)
        hs_ref[pl.ds(r, batch), :] = h
        return c, h

    UNROLL = 64

    def step(t, carry):
        c, h = carry
        base = (T - 1 - t * UNROLL) * batch
        for u in range(UNROLL):
            c, h = one_step(base - u * batch, c, h)
        return (c, h)

    c, h = jax.lax.fori_loop(0, T // UNROLL, step, (c_ref[:], h_ref[:]))
    c_ref[:] = c
    h_ref[:] = h

    out_ref[:] = hs_ref[:].reshape(T, batch, mem).transpose(1, 0, 2)

    @pl.when(k == G - 1)
    def _():
        cf_ref[:] = c
        hf_ref[:] = h


def kernel(trees, inputs, ioux_w, ioux_b, iouh_w, iouh_b, fx_w, fx_b, fh_w, fh_b):
    del trees
    b, n, in_dim = inputs.shape
    mem = fx_b.shape[0]
    wx = 0.5 * jnp.concatenate([ioux_w, fx_w], axis=0).T
    wh = 0.5 * jnp.concatenate([iouh_w, fh_w], axis=0).T
    bias = 0.5 * jnp.concatenate([ioux_b + iouh_b, fx_b + fh_b])[None, :]

    T = 128
    G = n // T
    body = functools.partial(_lstm_body, T=T, mem=mem, batch=b)
    o_states, c_fin, h_fin = pl.pallas_call(
        body,
        grid=(G,),
        in_specs=[
            pl.BlockSpec((b, T, in_dim), lambda k: (0, pl.num_programs(0) - 1 - k, 0)),
            pl.BlockSpec((in_dim, 4 * mem), lambda k: (0, 0)),
            pl.BlockSpec((mem, 4 * mem), lambda k: (0, 0)),
            pl.BlockSpec((1, 4 * mem), lambda k: (0, 0)),
        ],
        out_specs=[
            pl.BlockSpec((b, T, mem), lambda k: (0, pl.num_programs(0) - 1 - k, 0)),
            pl.BlockSpec((b, mem), lambda k: (0, 0)),
            pl.BlockSpec((b, mem), lambda k: (0, 0)),
        ],
        out_shape=[
            jax.ShapeDtypeStruct((b, n, mem), jnp.float32),
            jax.ShapeDtypeStruct((b, mem), jnp.float32),
            jax.ShapeDtypeStruct((b, mem), jnp.float32),
        ],
        scratch_shapes=[
            pltpu.VMEM((T * b, 4 * mem), jnp.float32),
            pltpu.VMEM((T * b, mem), jnp.float32),
            pltpu.VMEM((b, mem), jnp.float32),
            pltpu.VMEM((b, mem), jnp.float32),
        ],
    )(inputs, wx, wh, bias)
    return (o_states, c_fin, h_fin)
